# Initial kernel scaffold; baseline (speedup 1.0000x reference)
import jax
import jax.numpy as jnp
from jax.experimental import pallas as pl


def kernel(numbers, mapper, eye_matrix):
    raise NotImplementedError("write your pallas kernel here")


# speedup vs baseline: 13.0499x; 13.0499x over previous
import functools

import jax
import jax.numpy as jnp
from jax import lax
from jax.experimental import pallas as pl
from jax.experimental.pallas import tpu as pltpu
from jax.experimental.pallas import tpu_sc as plsc

N = 500000
D = 64
R = 800
NCHUNK = N // R
NC = 2
NS = 16
NW = NC * NS
TPW = -(-NCHUNK // NW)
MPAD = 128


def _body(numbers_hbm, mapper_hbm, eye_hbm, out_hbm, map_v, eye_v, z_v, c_v, buf):
    wid = lax.axis_index("s") * NC + lax.axis_index("c")
    lane = lax.broadcasted_iota(jnp.int32, (16,), 0)
    zeros16 = jnp.zeros((16,), jnp.float32)

    pltpu.sync_copy(mapper_hbm, map_v)
    pltpu.sync_copy(eye_hbm, eye_v)

    def zero_row(i, _):
        for k in range(D // 16):
            buf[i, pl.ds(k * 16, 16)] = zeros16
        return 0

    lax.fori_loop(0, R, zero_row, 0)

    def chunk_body(t, _):
        chunk = wid + t * NW

        @pl.when(chunk < NCHUNK)
        def _():
            base = chunk * R
            pltpu.sync_copy(numbers_hbm.at[pl.ds(base, R)], z_v)

            def fill(j, _):
                z = z_v[pl.ds(j * 16, 16)]
                c = plsc.load_gather(map_v, [z])
                row = j * 16 + lane
                val = plsc.load_gather(eye_v, [c, c])
                plsc.store_scatter(buf, [row, c], val)
                c_v[pl.ds(j * 16, 16)] = c
                return 0

            lax.fori_loop(0, R // 16, fill, 0)
            pltpu.sync_copy(buf, out_hbm.at[pl.ds(base, R)])

            def clear(j, _):
                c = c_v[pl.ds(j * 16, 16)]
                row = j * 16 + lane
                plsc.store_scatter(buf, [row, c], zeros16)
                return 0

            lax.fori_loop(0, R // 16, clear, 0)

        return 0

    lax.fori_loop(0, TPW, chunk_body, 0)


@jax.jit
def kernel(numbers, mapper, eye_matrix):
    mapper_p = jnp.zeros((MPAD,), jnp.int32).at[: mapper.shape[0]].set(mapper)
    run = functools.partial(
        pl.kernel,
        out_type=jax.ShapeDtypeStruct((N, D), jnp.float32),
        mesh=plsc.VectorSubcoreMesh(core_axis_name="c", subcore_axis_name="s"),
        compiler_params=pltpu.CompilerParams(needs_layout_passes=False),
        scratch_types=[
            pltpu.VMEM((MPAD,), jnp.int32),
            pltpu.VMEM((D, D), jnp.float32),
            pltpu.VMEM((R,), jnp.int32),
            pltpu.VMEM((R,), jnp.int32),
            pltpu.VMEM((R, D), jnp.float32),
        ],
    )(_body)
    return run(numbers, mapper_p, eye_matrix)
